# baseline (speedup 1.0000x reference)
import functools

import jax
import jax.numpy as jnp
from jax import lax
from jax.experimental import pallas as pl
from jax.experimental.pallas import tpu as pltpu
from jax.experimental.pallas import tpu_sc as plsc

_B = 4096
_D = 32
_V = 1000000
_NC = 2
_NS = 16
_NW = _NC * _NS
_BPW = _B // _NW
_NBLK = _BPW // 16
_EPW = 2
_NWAVE = _BPW // _EPW

_LN2 = 0.6931471805599453
_SQRT2 = 1.4142135623730951


def _log16(x):
  bits = plsc.bitcast(x, jnp.int32)
  e = (bits >> 23) - 127
  m = plsc.bitcast((bits & 0x007FFFFF) | 0x3F800000, jnp.float32)
  big = m > _SQRT2
  m = jnp.where(big, m * 0.5, m)
  e = jnp.where(big, e + 1, e)
  r = (m - 1.0) / (m + 1.0)
  r2 = r * r
  p = r2 * (1.0 / 3.0 + r2 * (1.0 / 5.0 + r2 * (1.0 / 7.0 + r2 * (1.0 / 9.0))))
  return 2.0 * r * (1.0 + p) + e.astype(jnp.float32) * _LN2


def _sc_body(i_hbm, j_hbm, co_hbm, w_hbm, wvt_hbm, wwt_hbm, bv_hbm, bw_hbm,
             out_hbm, idx_i, idx_j, co_v, w_v, buf_a, buf_b, buf_c, buf_d,
             buf_e, buf_f, buf_g, prod_v, bi_v, bj_v, part_v, sem_b, sem_a2,
             sem_b2, sem_c2, sem_d2, sem_e2, sem_f2, sem_g2):
  wid = lax.axis_index("s") * _NC + lax.axis_index("c")
  base = pl.multiple_of(wid * _BPW, _BPW)

  pltpu.sync_copy(i_hbm.at[pl.ds(base, _BPW)], idx_i)
  pltpu.sync_copy(j_hbm.at[pl.ds(base, _BPW)], idx_j)
  pltpu.sync_copy(co_hbm.at[pl.ds(base, _BPW)], co_v)
  pltpu.sync_copy(w_hbm.at[pl.ds(base, _BPW)], w_v)

  cp_bi = pltpu.async_copy(bv_hbm.at[0].at[idx_i], bi_v, sem_b)
  cp_bj = pltpu.async_copy(bw_hbm.at[0].at[idx_j], bj_v, sem_b)

  d16 = lax.iota(jnp.int32, 16)
  dbase = d16 * _BPW
  zero = jnp.zeros((16,), jnp.float32)
  zero_i = jnp.zeros((16,), jnp.int32)

  bufs = (buf_a, buf_b, buf_c, buf_d, buf_e, buf_f, buf_g)
  sems = (sem_a2, sem_b2, sem_c2, sem_d2, sem_e2, sem_f2, sem_g2)
  nbuf = len(bufs)

  def wave_scalars(w):
    g = (_EPW * w) // 16
    off = (_EPW * w) % 16
    ivec = idx_i[pl.ds(16 * g, 16)]
    jvec = idx_j[pl.ds(16 * g, 16)]
    return [(ivec[off + e], jvec[off + e]) for e in range(_EPW)]

  def fire(w):
    buf = bufs[w % nbuf]
    sem = sems[w % nbuf]
    cps = []
    for e, (vi, vj) in enumerate(wave_scalars(w)):
      tci = pl.multiple_of((vi // 128) * 128, 128)
      tcj = pl.multiple_of((vj // 128) * 128, 128)
      cps.append(pltpu.async_copy(
          wvt_hbm.at[:, pl.ds(tci, 128)],
          buf.at[pl.ds((2 * e) * _D, _D), :], sem))
      cps.append(pltpu.async_copy(
          wwt_hbm.at[:, pl.ds(tcj, 128)],
          buf.at[pl.ds((2 * e + 1) * _D, _D), :], sem))
    return cps

  def extract(w):
    buf = bufs[w % nbuf]
    for e, (vi, vj) in enumerate(wave_scalars(w)):
      c = _EPW * w + e
      li = zero_i + (vi % 128)
      lj = zero_i + (vj % 128)
      r_vi = d16 + (2 * e) * _D
      r_wj = d16 + (2 * e + 1) * _D
      gva = plsc.load_gather(buf, [r_vi, li])
      gvb = plsc.load_gather(buf, [r_vi + 16, li])
      gwa = plsc.load_gather(buf, [r_wj, lj])
      gwb = plsc.load_gather(buf, [r_wj + 16, lj])
      psum = gva * gwa + gvb * gwb
      plsc.store_scatter(prod_v, [dbase + c], psum)

  depth = nbuf - 1
  pending = [fire(w) for w in range(depth)]
  for w in range(_NWAVE):
    if w + depth < _NWAVE:
      nxt = fire(w + depth)
    else:
      nxt = []
    for cp in pending[0]:
      cp.wait()
    extract(w)
    pending = pending[1:] + [nxt]

  cp_bi.wait()
  cp_bj.wait()

  s1, s2, s3, t1, t2 = zero, zero, zero, zero, zero
  for k in range(_NBLK):
    acc = zero
    for dd in range(16):
      acc = acc + prod_v[pl.ds(dd * _BPW + 16 * k, 16)]
    cok = co_v[pl.ds(16 * k, 16)]
    wk = w_v[pl.ds(16 * k, 16)]
    a = acc - _log16(cok)
    wa = wk * a
    s1 = s1 + wa * a
    s2 = s2 + wa
    s3 = s3 + wk
    b = bi_v[pl.ds(16 * k, 16)] + bj_v[pl.ds(16 * k, 16)]
    t1 = t1 + b
    t2 = t2 + b * b

  part_v[0, pl.ds(0, 16)] = s1
  part_v[0, pl.ds(16, 16)] = s2
  part_v[0, pl.ds(32, 16)] = s3
  part_v[0, pl.ds(48, 16)] = t1
  part_v[0, pl.ds(64, 16)] = t2
  part_v[0, pl.ds(80, 16)] = zero
  part_v[0, pl.ds(96, 16)] = zero
  part_v[0, pl.ds(112, 16)] = zero
  pltpu.sync_copy(part_v, out_hbm.at[wid])


def _tc_body(p_ref, out_ref):
  p = p_ref[:, 0, :]
  s1 = jnp.sum(p[:, 0:16])
  s2 = jnp.sum(p[:, 16:32])
  s3 = jnp.sum(p[:, 32:48])
  t1 = jnp.sum(p[:, 48:64])
  t2 = jnp.sum(p[:, 64:80])
  out_ref[0, 0] = 0.5 * (_B * s1 + 2.0 * s2 * t1 + s3 * t2)


def kernel(i, j, co_occur, weight, Wv, Ww, bv, bw):
  mesh = plsc.VectorSubcoreMesh(
      core_axis_name="c", subcore_axis_name="s", num_cores=_NC,
      num_subcores=_NS)
  sc_call = functools.partial(
      pl.kernel,
      out_type=jax.ShapeDtypeStruct((_NW, 1, 128), jnp.float32),
      mesh=mesh,
      compiler_params=pltpu.CompilerParams(needs_layout_passes=False),
      scratch_types=[
          pltpu.VMEM((_BPW,), jnp.int32),
          pltpu.VMEM((_BPW,), jnp.int32),
          pltpu.VMEM((_BPW,), jnp.float32),
          pltpu.VMEM((_BPW,), jnp.float32),
          pltpu.VMEM((2 * _EPW * _D, 128), jnp.float32),
          pltpu.VMEM((2 * _EPW * _D, 128), jnp.float32),
          pltpu.VMEM((2 * _EPW * _D, 128), jnp.float32),
          pltpu.VMEM((2 * _EPW * _D, 128), jnp.float32),
          pltpu.VMEM((2 * _EPW * _D, 128), jnp.float32),
          pltpu.VMEM((2 * _EPW * _D, 128), jnp.float32),
          pltpu.VMEM((2 * _EPW * _D, 128), jnp.float32),
          pltpu.VMEM((16 * _BPW,), jnp.float32),
          pltpu.VMEM((_BPW,), jnp.float32),
          pltpu.VMEM((_BPW,), jnp.float32),
          pltpu.VMEM((1, 128), jnp.float32),
          pltpu.SemaphoreType.DMA,
          pltpu.SemaphoreType.DMA,
          pltpu.SemaphoreType.DMA,
          pltpu.SemaphoreType.DMA,
          pltpu.SemaphoreType.DMA,
          pltpu.SemaphoreType.DMA,
          pltpu.SemaphoreType.DMA,
          pltpu.SemaphoreType.DMA,
      ],
  )(_sc_body)
  partials = sc_call(i, j, co_occur, weight,
                     jnp.transpose(Wv), jnp.transpose(Ww),
                     jnp.transpose(bv), jnp.transpose(bw))

  out = pl.pallas_call(
      _tc_body,
      out_shape=jax.ShapeDtypeStruct((1, 1), jnp.float32),
      in_specs=[pl.BlockSpec(memory_space=pltpu.VMEM)],
      out_specs=pl.BlockSpec(memory_space=pltpu.SMEM),
  )(partials)
  return jnp.reshape(out, ())


# speedup vs baseline: 1.0196x; 1.0196x over previous
import functools

import jax
import jax.numpy as jnp
from jax import lax
from jax.experimental import pallas as pl
from jax.experimental.pallas import tpu as pltpu
from jax.experimental.pallas import tpu_sc as plsc

_B = 4096
_D = 32
_V = 1000000
_NC = 2
_NS = 16
_NW = _NC * _NS
_BPW = _B // _NW
_NBLK = _BPW // 16
_EPW = 2
_NWAVE = _BPW // _EPW

_LN2 = 0.6931471805599453
_SQRT2 = 1.4142135623730951


def _log16(x):
  bits = plsc.bitcast(x, jnp.int32)
  e = (bits >> 23) - 127
  m = plsc.bitcast((bits & 0x007FFFFF) | 0x3F800000, jnp.float32)
  big = m > _SQRT2
  m = jnp.where(big, m * 0.5, m)
  e = jnp.where(big, e + 1, e)
  r = (m - 1.0) / (m + 1.0)
  r2 = r * r
  p = r2 * (1.0 / 3.0 + r2 * (1.0 / 5.0 + r2 * (1.0 / 7.0 + r2 * (1.0 / 9.0))))
  return 2.0 * r * (1.0 + p) + e.astype(jnp.float32) * _LN2


def _sc_body(i_hbm, j_hbm, co_hbm, w_hbm, wvt_hbm, wwt_hbm, bv_hbm, bw_hbm,
             out_hbm, idx_i, idx_j, co_v, w_v, buf_a, buf_b, buf_c, buf_d,
             buf_e, buf_f, buf_g, prod_v, bi_v, bj_v, part_v, sem_b, sem_a2,
             sem_b2, sem_c2, sem_d2, sem_e2, sem_f2, sem_g2):
  wid = lax.axis_index("s") * _NC + lax.axis_index("c")
  base = pl.multiple_of(wid * _BPW, _BPW)

  cp_i = pltpu.async_copy(i_hbm.at[pl.ds(base, _BPW)], idx_i, sem_b)
  cp_j = pltpu.async_copy(j_hbm.at[pl.ds(base, _BPW)], idx_j, sem_b)
  cp_co = pltpu.async_copy(co_hbm.at[pl.ds(base, _BPW)], co_v, sem_b)
  cp_w = pltpu.async_copy(w_hbm.at[pl.ds(base, _BPW)], w_v, sem_b)
  cp_i.wait()
  cp_j.wait()

  cp_bi = pltpu.async_copy(bv_hbm.at[0].at[idx_i], bi_v, sem_b)
  cp_bj = pltpu.async_copy(bw_hbm.at[0].at[idx_j], bj_v, sem_b)

  d16 = lax.iota(jnp.int32, 16)
  dbase = d16 * _BPW
  zero = jnp.zeros((16,), jnp.float32)
  zero_i = jnp.zeros((16,), jnp.int32)

  bufs = (buf_a, buf_b, buf_c, buf_d, buf_e, buf_f, buf_g)
  sems = (sem_a2, sem_b2, sem_c2, sem_d2, sem_e2, sem_f2, sem_g2)
  nbuf = len(bufs)

  def wave_scalars(w):
    g = (_EPW * w) // 16
    off = (_EPW * w) % 16
    ivec = idx_i[pl.ds(16 * g, 16)]
    jvec = idx_j[pl.ds(16 * g, 16)]
    return [(ivec[off + e], jvec[off + e]) for e in range(_EPW)]

  def fire(w):
    buf = bufs[w % nbuf]
    sem = sems[w % nbuf]
    cps = []
    for e, (vi, vj) in enumerate(wave_scalars(w)):
      tci = pl.multiple_of((vi // 128) * 128, 128)
      tcj = pl.multiple_of((vj // 128) * 128, 128)
      cps.append(pltpu.async_copy(
          wvt_hbm.at[:, pl.ds(tci, 128)],
          buf.at[pl.ds((2 * e) * _D, _D), :], sem))
      cps.append(pltpu.async_copy(
          wwt_hbm.at[:, pl.ds(tcj, 128)],
          buf.at[pl.ds((2 * e + 1) * _D, _D), :], sem))
    return cps

  def extract(w):
    buf = bufs[w % nbuf]
    for e, (vi, vj) in enumerate(wave_scalars(w)):
      c = _EPW * w + e
      li = zero_i + (vi % 128)
      lj = zero_i + (vj % 128)
      r_vi = d16 + (2 * e) * _D
      r_wj = d16 + (2 * e + 1) * _D
      gva = plsc.load_gather(buf, [r_vi, li])
      gvb = plsc.load_gather(buf, [r_vi + 16, li])
      gwa = plsc.load_gather(buf, [r_wj, lj])
      gwb = plsc.load_gather(buf, [r_wj + 16, lj])
      psum = gva * gwa + gvb * gwb
      plsc.store_scatter(prod_v, [dbase + c], psum)

  depth = nbuf - 1
  pending = [fire(w) for w in range(depth)]
  for w in range(_NWAVE):
    if w + depth < _NWAVE:
      nxt = fire(w + depth)
    else:
      nxt = []
    for cp in pending[0]:
      cp.wait()
    extract(w)
    pending = pending[1:] + [nxt]

  cp_co.wait()
  cp_w.wait()
  cp_bi.wait()
  cp_bj.wait()

  s1, s2, s3, t1, t2 = zero, zero, zero, zero, zero
  for k in range(_NBLK):
    acc = zero
    for dd in range(16):
      acc = acc + prod_v[pl.ds(dd * _BPW + 16 * k, 16)]
    cok = co_v[pl.ds(16 * k, 16)]
    wk = w_v[pl.ds(16 * k, 16)]
    a = acc - _log16(cok)
    wa = wk * a
    s1 = s1 + wa * a
    s2 = s2 + wa
    s3 = s3 + wk
    b = bi_v[pl.ds(16 * k, 16)] + bj_v[pl.ds(16 * k, 16)]
    t1 = t1 + b
    t2 = t2 + b * b

  part_v[0, pl.ds(0, 16)] = s1
  part_v[0, pl.ds(16, 16)] = s2
  part_v[0, pl.ds(32, 16)] = s3
  part_v[0, pl.ds(48, 16)] = t1
  part_v[0, pl.ds(64, 16)] = t2
  part_v[0, pl.ds(80, 16)] = zero
  part_v[0, pl.ds(96, 16)] = zero
  part_v[0, pl.ds(112, 16)] = zero
  pltpu.sync_copy(part_v, out_hbm.at[wid])


def _tc_body(p_ref, out_ref):
  p = p_ref[:, 0, :]
  s1 = jnp.sum(p[:, 0:16])
  s2 = jnp.sum(p[:, 16:32])
  s3 = jnp.sum(p[:, 32:48])
  t1 = jnp.sum(p[:, 48:64])
  t2 = jnp.sum(p[:, 64:80])
  out_ref[0, 0] = 0.5 * (_B * s1 + 2.0 * s2 * t1 + s3 * t2)


def kernel(i, j, co_occur, weight, Wv, Ww, bv, bw):
  mesh = plsc.VectorSubcoreMesh(
      core_axis_name="c", subcore_axis_name="s", num_cores=_NC,
      num_subcores=_NS)
  sc_call = functools.partial(
      pl.kernel,
      out_type=jax.ShapeDtypeStruct((_NW, 1, 128), jnp.float32),
      mesh=mesh,
      compiler_params=pltpu.CompilerParams(needs_layout_passes=False),
      scratch_types=[
          pltpu.VMEM((_BPW,), jnp.int32),
          pltpu.VMEM((_BPW,), jnp.int32),
          pltpu.VMEM((_BPW,), jnp.float32),
          pltpu.VMEM((_BPW,), jnp.float32),
          pltpu.VMEM((2 * _EPW * _D, 128), jnp.float32),
          pltpu.VMEM((2 * _EPW * _D, 128), jnp.float32),
          pltpu.VMEM((2 * _EPW * _D, 128), jnp.float32),
          pltpu.VMEM((2 * _EPW * _D, 128), jnp.float32),
          pltpu.VMEM((2 * _EPW * _D, 128), jnp.float32),
          pltpu.VMEM((2 * _EPW * _D, 128), jnp.float32),
          pltpu.VMEM((2 * _EPW * _D, 128), jnp.float32),
          pltpu.VMEM((16 * _BPW,), jnp.float32),
          pltpu.VMEM((_BPW,), jnp.float32),
          pltpu.VMEM((_BPW,), jnp.float32),
          pltpu.VMEM((1, 128), jnp.float32),
          pltpu.SemaphoreType.DMA,
          pltpu.SemaphoreType.DMA,
          pltpu.SemaphoreType.DMA,
          pltpu.SemaphoreType.DMA,
          pltpu.SemaphoreType.DMA,
          pltpu.SemaphoreType.DMA,
          pltpu.SemaphoreType.DMA,
          pltpu.SemaphoreType.DMA,
      ],
  )(_sc_body)
  partials = sc_call(i, j, co_occur, weight,
                     jnp.transpose(Wv), jnp.transpose(Ww),
                     jnp.transpose(bv), jnp.transpose(bw))

  out = pl.pallas_call(
      _tc_body,
      out_shape=jax.ShapeDtypeStruct((1, 1), jnp.float32),
      in_specs=[pl.BlockSpec(memory_space=pltpu.VMEM)],
      out_specs=pl.BlockSpec(memory_space=pltpu.SMEM),
  )(partials)
  return jnp.reshape(out, ())
